# baseline (speedup 1.0000x reference)
import functools

import jax
import jax.numpy as jnp
from jax import lax
from jax.experimental import pallas as pl
from jax.experimental.pallas import tpu as pltpu
from jax.experimental.pallas import tpu_sc as plsc

D = 100
DP = 112
NCHUNK = 7
CHUNK_OFF = (0, 16, 32, 48, 64, 80, 84)
B = 4096
L = 200
NW = 32
BPW = B // NW
IPW = BPW * L
SPLIT0 = 96
SPLIT1 = L - SPLIT0


def _issue(b, buf, sem, tab_hbm, idx_v):
    off = b * L
    pltpu.make_async_copy(
        tab_hbm.at[idx_v.at[pl.ds(off, SPLIT0)]],
        buf.at[pl.ds(0, SPLIT0)], sem).start()
    pltpu.make_async_copy(
        tab_hbm.at[idx_v.at[pl.ds(off + SPLIT0, SPLIT1)]],
        buf.at[pl.ds(SPLIT0, SPLIT1)], sem).start()


def _wait(b, buf, sem, tab_hbm, idx_v):
    off = b * L
    pltpu.make_async_copy(
        tab_hbm.at[idx_v.at[pl.ds(off, SPLIT0)]],
        buf.at[pl.ds(0, SPLIT0)], sem).wait()
    pltpu.make_async_copy(
        tab_hbm.at[idx_v.at[pl.ds(off + SPLIT0, SPLIT1)]],
        buf.at[pl.ds(SPLIT0, SPLIT1)], sem).wait()


def _accum_and_max(buf, macc):
    def lbody(l, accs):
        return tuple(a + buf[l, pl.ds(CHUNK_OFF[ci], 16)]
                     for ci, a in enumerate(accs))
    zero = jnp.zeros((16,), jnp.float32)
    accs = lax.fori_loop(0, L, lbody, (zero,) * NCHUNK)
    for ci in range(NCHUNK):
        sl = pl.ds(ci * 16, 16)
        macc[sl] = jnp.maximum(macc[sl], accs[ci])


@functools.partial(
    pl.kernel,
    out_type=jax.ShapeDtypeStruct((NW, DP), jnp.float32),
    mesh=plsc.VectorSubcoreMesh(core_axis_name="c", subcore_axis_name="s"),
    compiler_params=pltpu.CompilerParams(use_tc_tiling_on_sc=False),
    scratch_types=[
        pltpu.VMEM((IPW,), jnp.int32),
        pltpu.VMEM((L, D), jnp.float32),
        pltpu.VMEM((L, D), jnp.float32),
        pltpu.VMEM((DP,), jnp.float32),
        pltpu.SemaphoreType.DMA,
        pltpu.SemaphoreType.DMA,
    ],
)
def _sc_gather_sum_max(xf_hbm, tab_hbm, part_hbm,
                       idx_v, buf0, buf1, macc, sem0, sem1):
    wid = lax.axis_index("s") * 2 + lax.axis_index("c")
    pltpu.sync_copy(xf_hbm.at[pl.ds(wid * IPW, IPW)], idx_v)

    neg = jnp.full((16,), -1e30, jnp.float32)
    for ci in range(NCHUNK):
        macc[pl.ds(ci * 16, 16)] = neg

    _issue(0, buf0, sem0, tab_hbm, idx_v)
    _issue(1, buf1, sem1, tab_hbm, idx_v)

    def pbody(p, carry):
        b0 = 2 * p
        _wait(b0, buf0, sem0, tab_hbm, idx_v)
        _accum_and_max(buf0, macc)

        @pl.when(p < BPW // 2 - 1)
        def _():
            _issue(b0 + 2, buf0, sem0, tab_hbm, idx_v)

        _wait(b0 + 1, buf1, sem1, tab_hbm, idx_v)
        _accum_and_max(buf1, macc)

        @pl.when(p < BPW // 2 - 1)
        def _():
            _issue(b0 + 3, buf1, sem1, tab_hbm, idx_v)

        return carry

    lax.fori_loop(0, BPW // 2, pbody, 0)
    pltpu.sync_copy(macc, part_hbm.at[wid])


def _tc_classifier_body(part_ref, w_ref, b_ref, y_ref, pred_ref, loss_ref):
    part = part_ref[...]
    pm = jnp.max(part, axis=0, keepdims=True)
    pm100 = jnp.concatenate([pm[:, :96], pm[:, 108:112]], axis=1)
    pm100 = jnp.where(pm100 >= 0, pm100, 0.01 * pm100)
    logits = jnp.sum(w_ref[...] * pm100, axis=1, keepdims=True) + b_ref[...]
    m = jnp.max(logits, axis=0, keepdims=True)
    e = jnp.exp(logits - m)
    pred = e / jnp.sum(e, axis=0, keepdims=True)
    pred_ref[...] = pred
    pm2 = jnp.max(pred, axis=0, keepdims=True)
    e2 = jnp.exp(pred - pm2)
    lse = jnp.log(jnp.sum(e2, axis=0, keepdims=True)) + pm2
    log_probs = pred - lse
    loss_ref[...] = -jnp.sum(y_ref[...] * log_probs, axis=0, keepdims=True)


_tc_classifier = pl.pallas_call(
    _tc_classifier_body,
    out_shape=(jax.ShapeDtypeStruct((4, 1), jnp.float32),
               jax.ShapeDtypeStruct((1, 1), jnp.float32)),
)


def kernel(x, y, E_td, w_out, b_out):
    part = _sc_gather_sum_max(x.reshape(-1), E_td)
    pred2, loss2 = _tc_classifier(part, w_out, b_out[:, None], y[:, None])
    return pred2[:, 0], loss2[0, 0]


# speedup vs baseline: 1.2855x; 1.2855x over previous
import functools

import jax
import jax.numpy as jnp
from jax import lax
from jax.experimental import pallas as pl
from jax.experimental.pallas import tpu as pltpu
from jax.experimental.pallas import tpu_sc as plsc

D = 100
DP = 128
NCHUNK = 8
B = 4096
L = 200
NW = 32
BPW = B // NW
IPW = BPW * L
SPLIT0 = 96
SPLIT1 = L - SPLIT0


def _issue(b, buf, sem, tab_hbm, idx_v):
    off = b * L
    pltpu.make_async_copy(
        tab_hbm.at[idx_v.at[pl.ds(off, SPLIT0)]],
        buf.at[pl.ds(0, SPLIT0)], sem).start()
    pltpu.make_async_copy(
        tab_hbm.at[idx_v.at[pl.ds(off + SPLIT0, SPLIT1)]],
        buf.at[pl.ds(SPLIT0, SPLIT1)], sem).start()


def _wait(b, buf, sem, tab_hbm, idx_v):
    off = b * L
    pltpu.make_async_copy(
        tab_hbm.at[idx_v.at[pl.ds(off, SPLIT0)]],
        buf.at[pl.ds(0, SPLIT0)], sem).wait()
    pltpu.make_async_copy(
        tab_hbm.at[idx_v.at[pl.ds(off + SPLIT0, SPLIT1)]],
        buf.at[pl.ds(SPLIT0, SPLIT1)], sem).wait()


def _accum_and_max(buf, macc):
    def lbody(l, accs):
        return tuple(a + buf[l, pl.ds(ci * 16, 16)]
                     for ci, a in enumerate(accs))
    zero = jnp.zeros((16,), jnp.float32)
    accs = lax.fori_loop(0, L, lbody, (zero,) * NCHUNK)
    for ci in range(NCHUNK):
        sl = pl.ds(ci * 16, 16)
        macc[sl] = jnp.maximum(macc[sl], accs[ci])


@functools.partial(
    pl.kernel,
    out_type=jax.ShapeDtypeStruct((NW, DP), jnp.float32),
    mesh=plsc.VectorSubcoreMesh(core_axis_name="c", subcore_axis_name="s"),
    scratch_types=[
        pltpu.VMEM((IPW,), jnp.int32),
        pltpu.VMEM((L, DP), jnp.float32),
        pltpu.VMEM((L, DP), jnp.float32),
        pltpu.VMEM((DP,), jnp.float32),
        pltpu.SemaphoreType.DMA,
        pltpu.SemaphoreType.DMA,
    ],
)
def _sc_gather_sum_max(xf_hbm, tab_hbm, part_hbm,
                       idx_v, buf0, buf1, macc, sem0, sem1):
    wid = lax.axis_index("s") * 2 + lax.axis_index("c")
    pltpu.sync_copy(xf_hbm.at[pl.ds(wid * IPW, IPW)], idx_v)

    neg = jnp.full((16,), -1e30, jnp.float32)
    for ci in range(NCHUNK):
        macc[pl.ds(ci * 16, 16)] = neg

    _issue(0, buf0, sem0, tab_hbm, idx_v)
    _issue(1, buf1, sem1, tab_hbm, idx_v)

    def pbody(p, carry):
        b0 = 2 * p
        _wait(b0, buf0, sem0, tab_hbm, idx_v)
        _accum_and_max(buf0, macc)

        @pl.when(p < BPW // 2 - 1)
        def _():
            _issue(b0 + 2, buf0, sem0, tab_hbm, idx_v)

        _wait(b0 + 1, buf1, sem1, tab_hbm, idx_v)
        _accum_and_max(buf1, macc)

        @pl.when(p < BPW // 2 - 1)
        def _():
            _issue(b0 + 3, buf1, sem1, tab_hbm, idx_v)

        return carry

    lax.fori_loop(0, BPW // 2, pbody, 0)
    pltpu.sync_copy(macc, part_hbm.at[wid])


def _tc_classifier_body(part_ref, w_ref, b_ref, y_ref, pred_ref, loss_ref):
    part = part_ref[...]
    pm = jnp.max(part, axis=0, keepdims=True)
    pm100 = pm[:, :D]
    pm100 = jnp.where(pm100 >= 0, pm100, 0.01 * pm100)
    logits = jnp.sum(w_ref[...] * pm100, axis=1, keepdims=True) + b_ref[...]
    m = jnp.max(logits, axis=0, keepdims=True)
    e = jnp.exp(logits - m)
    pred = e / jnp.sum(e, axis=0, keepdims=True)
    pred_ref[...] = pred
    pm2 = jnp.max(pred, axis=0, keepdims=True)
    e2 = jnp.exp(pred - pm2)
    lse = jnp.log(jnp.sum(e2, axis=0, keepdims=True)) + pm2
    log_probs = pred - lse
    loss_ref[...] = -jnp.sum(y_ref[...] * log_probs, axis=0, keepdims=True)


_tc_classifier = pl.pallas_call(
    _tc_classifier_body,
    out_shape=(jax.ShapeDtypeStruct((4, 1), jnp.float32),
               jax.ShapeDtypeStruct((1, 1), jnp.float32)),
)


def kernel(x, y, E_td, w_out, b_out):
    tab = jnp.pad(E_td, ((0, 0), (0, DP - D)))
    part = _sc_gather_sum_max(x.reshape(-1), tab)
    pred2, loss2 = _tc_classifier(part, w_out, b_out[:, None], y[:, None])
    return pred2[:, 0], loss2[0, 0]


# speedup vs baseline: 4.7127x; 3.6660x over previous
import functools

import jax
import jax.numpy as jnp
from jax import lax
from jax.experimental import pallas as pl
from jax.experimental.pallas import tpu as pltpu
from jax.experimental.pallas import tpu_sc as plsc

D = 100
DP = 128
NCHUNK = 8
B = 4096
L = 200
NW = 32
BPW = B // NW
IPW = BPW * L
SPLIT0 = 96
SPLIT1 = L - SPLIT0


def _issue(b, buf, sem, tab_hbm, idx_v):
    off = b * L
    pltpu.make_async_copy(
        tab_hbm.at[idx_v.at[pl.ds(off, SPLIT0)]],
        buf.at[pl.ds(0, SPLIT0)], sem).start()
    pltpu.make_async_copy(
        tab_hbm.at[idx_v.at[pl.ds(off + SPLIT0, SPLIT1)]],
        buf.at[pl.ds(SPLIT0, SPLIT1)], sem).start()


def _wait(b, buf, sem, tab_hbm, idx_v):
    off = b * L
    pltpu.make_async_copy(
        tab_hbm.at[idx_v.at[pl.ds(off, SPLIT0)]],
        buf.at[pl.ds(0, SPLIT0)], sem).wait()
    pltpu.make_async_copy(
        tab_hbm.at[idx_v.at[pl.ds(off + SPLIT0, SPLIT1)]],
        buf.at[pl.ds(SPLIT0, SPLIT1)], sem).wait()


def _accum_and_max(buf, macc):
    def lbody(l, accs):
        return tuple(a + buf[l, pl.ds(ci * 16, 16)]
                     for ci, a in enumerate(accs))
    zero = jnp.zeros((16,), jnp.float32)
    accs = lax.fori_loop(0, L, lbody, (zero,) * NCHUNK)
    for ci in range(NCHUNK):
        sl = pl.ds(ci * 16, 16)
        macc[sl] = jnp.maximum(macc[sl], accs[ci])


@functools.partial(
    pl.kernel,
    out_type=jax.ShapeDtypeStruct((NW, DP), jnp.float32),
    mesh=plsc.VectorSubcoreMesh(core_axis_name="c", subcore_axis_name="s"),
    scratch_types=[
        pltpu.VMEM((IPW,), jnp.int32),
        pltpu.VMEM((L, DP), jnp.float32),
        pltpu.VMEM((L, DP), jnp.float32),
        pltpu.VMEM((DP,), jnp.float32),
        pltpu.SemaphoreType.DMA,
        pltpu.SemaphoreType.DMA,
    ],
)
def _sc_gather_sum_max(xf_hbm, tab_hbm, part_hbm,
                       idx_v, buf0, buf1, macc, sem0, sem1):
    wid = lax.axis_index("s") * 2 + lax.axis_index("c")
    pltpu.sync_copy(xf_hbm.at[pl.ds(wid * IPW, IPW)], idx_v)

    neg = jnp.full((16,), -1e30, jnp.float32)
    for ci in range(NCHUNK):
        macc[pl.ds(ci * 16, 16)] = neg

    _issue(0, buf0, sem0, tab_hbm, idx_v)
    _issue(1, buf1, sem1, tab_hbm, idx_v)

    def pbody(p, carry):
        b0 = 2 * p
        _wait(b0, buf0, sem0, tab_hbm, idx_v)
        _accum_and_max(buf0, macc)

        @pl.when(p < BPW // 2 - 1)
        def _():
            _issue(b0 + 2, buf0, sem0, tab_hbm, idx_v)

        _wait(b0 + 1, buf1, sem1, tab_hbm, idx_v)
        _accum_and_max(buf1, macc)

        @pl.when(p < BPW // 2 - 1)
        def _():
            _issue(b0 + 3, buf1, sem1, tab_hbm, idx_v)

        return carry

    lax.fori_loop(0, BPW // 2, pbody, 0)
    pltpu.sync_copy(macc, part_hbm.at[wid])


VBLK = 4096
VOCAB = 1000000


def _tc_transpose_body(tin_ref, tout_ref):
    t = tin_ref[...].T
    tout_ref[...] = jnp.concatenate(
        [t, jnp.zeros((VBLK, DP - D), jnp.float32)], axis=1)


_tc_transpose = pl.pallas_call(
    _tc_transpose_body,
    grid=(pl.cdiv(VOCAB, VBLK),),
    in_specs=[pl.BlockSpec((D, VBLK), lambda g: (0, g))],
    out_specs=pl.BlockSpec((VBLK, DP), lambda g: (g, 0)),
    out_shape=jax.ShapeDtypeStruct((VOCAB, DP), jnp.float32),
)


def _tc_classifier_body(part_ref, w_ref, b_ref, y_ref, pred_ref, loss_ref):
    part = part_ref[...]
    pm = jnp.max(part, axis=0, keepdims=True)
    pm100 = pm[:, :D]
    pm100 = jnp.where(pm100 >= 0, pm100, 0.01 * pm100)
    logits = jnp.sum(w_ref[...] * pm100, axis=1, keepdims=True) + b_ref[...]
    m = jnp.max(logits, axis=0, keepdims=True)
    e = jnp.exp(logits - m)
    pred = e / jnp.sum(e, axis=0, keepdims=True)
    pred_ref[...] = pred
    pm2 = jnp.max(pred, axis=0, keepdims=True)
    e2 = jnp.exp(pred - pm2)
    lse = jnp.log(jnp.sum(e2, axis=0, keepdims=True)) + pm2
    log_probs = pred - lse
    loss_ref[...] = -jnp.sum(y_ref[...] * log_probs, axis=0, keepdims=True)


_tc_classifier = pl.pallas_call(
    _tc_classifier_body,
    out_shape=(jax.ShapeDtypeStruct((4, 1), jnp.float32),
               jax.ShapeDtypeStruct((1, 1), jnp.float32)),
)


def kernel(x, y, E_td, w_out, b_out):
    tab = _tc_transpose(E_td.T)
    part = _sc_gather_sum_max(x.reshape(-1), tab)
    pred2, loss2 = _tc_classifier(part, w_out, b_out[:, None], y[:, None])
    return pred2[:, 0], loss2[0, 0]


# speedup vs baseline: 5.2883x; 1.1221x over previous
import functools

import jax
import jax.numpy as jnp
from jax import lax
from jax.experimental import pallas as pl
from jax.experimental.pallas import tpu as pltpu
from jax.experimental.pallas import tpu_sc as plsc

D = 100
DP = 128
NCHUNK = 8
B = 4096
L = 200
NW = 32
BPW = B // NW
IPW = BPW * L
SPLIT0 = 96
SPLIT1 = L - SPLIT0


def _issue(b, buf, sem, tab_hbm, idx_v):
    off = b * L
    pltpu.make_async_copy(
        tab_hbm.at[idx_v.at[pl.ds(off, SPLIT0)]],
        buf.at[pl.ds(0, SPLIT0)], sem).start()
    pltpu.make_async_copy(
        tab_hbm.at[idx_v.at[pl.ds(off + SPLIT0, SPLIT1)]],
        buf.at[pl.ds(SPLIT0, SPLIT1)], sem).start()


def _wait(b, buf, sem, tab_hbm, idx_v):
    off = b * L
    pltpu.make_async_copy(
        tab_hbm.at[idx_v.at[pl.ds(off, SPLIT0)]],
        buf.at[pl.ds(0, SPLIT0)], sem).wait()
    pltpu.make_async_copy(
        tab_hbm.at[idx_v.at[pl.ds(off + SPLIT0, SPLIT1)]],
        buf.at[pl.ds(SPLIT0, SPLIT1)], sem).wait()


def _accum_and_max(buf, macc):
    def lbody(l, accs):
        return tuple(a + buf[l, pl.ds(ci * 16, 16)]
                     for ci, a in enumerate(accs))
    zero = jnp.zeros((16,), jnp.float32)
    accs = lax.fori_loop(0, L, lbody, (zero,) * NCHUNK)
    for ci in range(NCHUNK):
        sl = pl.ds(ci * 16, 16)
        macc[sl] = jnp.maximum(macc[sl], accs[ci])


@functools.partial(
    pl.kernel,
    out_type=jax.ShapeDtypeStruct((NW, DP), jnp.float32),
    mesh=plsc.VectorSubcoreMesh(core_axis_name="c", subcore_axis_name="s"),
    scratch_types=[
        pltpu.VMEM((IPW,), jnp.int32),
        pltpu.VMEM((L, DP), jnp.float32),
        pltpu.VMEM((L, DP), jnp.float32),
        pltpu.VMEM((DP,), jnp.float32),
        pltpu.SemaphoreType.DMA,
        pltpu.SemaphoreType.DMA,
    ],
)
def _sc_gather_sum_max(xf_hbm, tab_hbm, part_hbm,
                       idx_v, buf0, buf1, macc, sem0, sem1):
    wid = lax.axis_index("s") * 2 + lax.axis_index("c")
    pltpu.sync_copy(xf_hbm.at[pl.ds(wid * IPW, IPW)], idx_v)

    neg = jnp.full((16,), -1e30, jnp.float32)
    for ci in range(NCHUNK):
        macc[pl.ds(ci * 16, 16)] = neg

    _issue(0, buf0, sem0, tab_hbm, idx_v)
    _issue(1, buf1, sem1, tab_hbm, idx_v)

    def pbody(p, carry):
        b0 = 2 * p
        _wait(b0, buf0, sem0, tab_hbm, idx_v)
        _accum_and_max(buf0, macc)

        @pl.when(p < BPW // 2 - 1)
        def _():
            _issue(b0 + 2, buf0, sem0, tab_hbm, idx_v)

        _wait(b0 + 1, buf1, sem1, tab_hbm, idx_v)
        _accum_and_max(buf1, macc)

        @pl.when(p < BPW // 2 - 1)
        def _():
            _issue(b0 + 3, buf1, sem1, tab_hbm, idx_v)

        return carry

    lax.fori_loop(0, BPW // 2, pbody, 0)
    pltpu.sync_copy(macc, part_hbm.at[wid])


VBLK = 8192
VOCAB = 1000000


def _tc_transpose_body(tin_ref, tout_ref):
    tout_ref[:, :D] = tin_ref[...].T


_tc_transpose = pl.pallas_call(
    _tc_transpose_body,
    grid=(pl.cdiv(VOCAB, VBLK),),
    in_specs=[pl.BlockSpec((D, VBLK), lambda g: (0, g))],
    out_specs=pl.BlockSpec((VBLK, DP), lambda g: (g, 0)),
    out_shape=jax.ShapeDtypeStruct((VOCAB, DP), jnp.float32),
)


def _tc_classifier_body(part_ref, w_ref, b_ref, y_ref, pred_ref, loss_ref):
    part = part_ref[...]
    pm = jnp.max(part, axis=0, keepdims=True)
    pm100 = pm[:, :D]
    pm100 = jnp.where(pm100 >= 0, pm100, 0.01 * pm100)
    logits = jnp.sum(w_ref[...] * pm100, axis=1, keepdims=True) + b_ref[...]
    m = jnp.max(logits, axis=0, keepdims=True)
    e = jnp.exp(logits - m)
    pred = e / jnp.sum(e, axis=0, keepdims=True)
    pred_ref[...] = pred
    pm2 = jnp.max(pred, axis=0, keepdims=True)
    e2 = jnp.exp(pred - pm2)
    lse = jnp.log(jnp.sum(e2, axis=0, keepdims=True)) + pm2
    log_probs = pred - lse
    loss_ref[...] = -jnp.sum(y_ref[...] * log_probs, axis=0, keepdims=True)


_tc_classifier = pl.pallas_call(
    _tc_classifier_body,
    out_shape=(jax.ShapeDtypeStruct((4, 1), jnp.float32),
               jax.ShapeDtypeStruct((1, 1), jnp.float32)),
)


def kernel(x, y, E_td, w_out, b_out):
    tab = _tc_transpose(E_td.T)
    part = _sc_gather_sum_max(x.reshape(-1), tab)
    pred2, loss2 = _tc_classifier(part, w_out, b_out[:, None], y[:, None])
    return pred2[:, 0], loss2[0, 0]


# speedup vs baseline: 5.4037x; 1.0218x over previous
import functools

import jax
import jax.numpy as jnp
from jax import lax
from jax.experimental import pallas as pl
from jax.experimental.pallas import tpu as pltpu
from jax.experimental.pallas import tpu_sc as plsc

D = 100
DP = 128
NCHUNK = 8
B = 4096
L = 200
NW = 32
BPW = B // NW
IPW = BPW * L
SPLIT0 = 96
SPLIT1 = L - SPLIT0


def _issue(b, buf, sem, tab_hbm, idx_v):
    off = b * L
    pltpu.make_async_copy(
        tab_hbm.at[idx_v.at[pl.ds(off, SPLIT0)]],
        buf.at[pl.ds(0, SPLIT0)], sem).start()
    pltpu.make_async_copy(
        tab_hbm.at[idx_v.at[pl.ds(off + SPLIT0, SPLIT1)]],
        buf.at[pl.ds(SPLIT0, SPLIT1)], sem).start()


def _wait(b, buf, sem, tab_hbm, idx_v):
    off = b * L
    pltpu.make_async_copy(
        tab_hbm.at[idx_v.at[pl.ds(off, SPLIT0)]],
        buf.at[pl.ds(0, SPLIT0)], sem).wait()
    pltpu.make_async_copy(
        tab_hbm.at[idx_v.at[pl.ds(off + SPLIT0, SPLIT1)]],
        buf.at[pl.ds(SPLIT0, SPLIT1)], sem).wait()


def _accum_and_max(buf, macc):
    def lbody(l, accs):
        return tuple(a + buf[l, pl.ds(ci * 16, 16)]
                     for ci, a in enumerate(accs))
    zero = jnp.zeros((16,), jnp.float32)
    accs = lax.fori_loop(0, L, lbody, (zero,) * NCHUNK)
    for ci in range(NCHUNK):
        sl = pl.ds(ci * 16, 16)
        macc[sl] = jnp.maximum(macc[sl], accs[ci])


@functools.partial(
    pl.kernel,
    out_type=jax.ShapeDtypeStruct((NW, DP), jnp.float32),
    mesh=plsc.VectorSubcoreMesh(core_axis_name="c", subcore_axis_name="s"),
    scratch_types=[
        pltpu.VMEM((IPW,), jnp.int32),
        pltpu.VMEM((L, DP), jnp.float32),
        pltpu.VMEM((L, DP), jnp.float32),
        pltpu.VMEM((DP,), jnp.float32),
        pltpu.SemaphoreType.DMA,
        pltpu.SemaphoreType.DMA,
    ],
)
def _sc_gather_sum_max(xf_hbm, tab_hbm, part_hbm,
                       idx_v, buf0, buf1, macc, sem0, sem1):
    wid = lax.axis_index("s") * 2 + lax.axis_index("c")
    pltpu.sync_copy(xf_hbm.at[pl.ds(wid * IPW, IPW)], idx_v)

    neg = jnp.full((16,), -1e30, jnp.float32)
    for ci in range(NCHUNK):
        macc[pl.ds(ci * 16, 16)] = neg

    _issue(0, buf0, sem0, tab_hbm, idx_v)
    _issue(1, buf1, sem1, tab_hbm, idx_v)

    def pbody(p, carry):
        b0 = 2 * p
        _wait(b0, buf0, sem0, tab_hbm, idx_v)
        _accum_and_max(buf0, macc)

        @pl.when(p < BPW // 2 - 1)
        def _():
            _issue(b0 + 2, buf0, sem0, tab_hbm, idx_v)

        _wait(b0 + 1, buf1, sem1, tab_hbm, idx_v)
        _accum_and_max(buf1, macc)

        @pl.when(p < BPW // 2 - 1)
        def _():
            _issue(b0 + 3, buf1, sem1, tab_hbm, idx_v)

        return carry

    lax.fori_loop(0, BPW // 2, pbody, 0)
    pltpu.sync_copy(macc, part_hbm.at[wid])


VBLK = 16384
VOCAB = 1000000


def _tc_transpose_body(tin_ref, tout_ref):
    tout_ref[:, :D] = tin_ref[...].T


_tc_transpose = pl.pallas_call(
    _tc_transpose_body,
    grid=(pl.cdiv(VOCAB, VBLK),),
    in_specs=[pl.BlockSpec((D, VBLK), lambda g: (0, g))],
    out_specs=pl.BlockSpec((VBLK, DP), lambda g: (g, 0)),
    out_shape=jax.ShapeDtypeStruct((VOCAB, DP), jnp.float32),
)


def _tc_classifier_body(part_ref, w_ref, b_ref, y_ref, pred_ref, loss_ref):
    part = part_ref[...]
    pm = jnp.max(part, axis=0, keepdims=True)
    pm100 = pm[:, :D]
    pm100 = jnp.where(pm100 >= 0, pm100, 0.01 * pm100)
    logits = jnp.sum(w_ref[...] * pm100, axis=1, keepdims=True) + b_ref[...]
    m = jnp.max(logits, axis=0, keepdims=True)
    e = jnp.exp(logits - m)
    pred = e / jnp.sum(e, axis=0, keepdims=True)
    pred_ref[...] = pred
    pm2 = jnp.max(pred, axis=0, keepdims=True)
    e2 = jnp.exp(pred - pm2)
    lse = jnp.log(jnp.sum(e2, axis=0, keepdims=True)) + pm2
    log_probs = pred - lse
    loss_ref[...] = -jnp.sum(y_ref[...] * log_probs, axis=0, keepdims=True)


_tc_classifier = pl.pallas_call(
    _tc_classifier_body,
    out_shape=(jax.ShapeDtypeStruct((4, 1), jnp.float32),
               jax.ShapeDtypeStruct((1, 1), jnp.float32)),
)


def kernel(x, y, E_td, w_out, b_out):
    tab = _tc_transpose(E_td.T)
    part = _sc_gather_sum_max(x.reshape(-1), tab)
    pred2, loss2 = _tc_classifier(part, w_out, b_out[:, None], y[:, None])
    return pred2[:, 0], loss2[0, 0]


# speedup vs baseline: 5.4054x; 1.0003x over previous
import functools

import jax
import jax.numpy as jnp
from jax import lax
from jax.experimental import pallas as pl
from jax.experimental.pallas import tpu as pltpu
from jax.experimental.pallas import tpu_sc as plsc

D = 100
DP = 128
NCHUNK = 8
B = 4096
L = 200
NW = 32
BPW = B // NW
IPW = BPW * L
SPLIT0 = 96
SPLIT1 = L - SPLIT0


def _issue(b, buf, sem, tab_hbm, idx_v):
    off = b * L
    pltpu.make_async_copy(
        tab_hbm.at[idx_v.at[pl.ds(off, SPLIT0)]],
        buf.at[pl.ds(0, SPLIT0)], sem).start()
    pltpu.make_async_copy(
        tab_hbm.at[idx_v.at[pl.ds(off + SPLIT0, SPLIT1)]],
        buf.at[pl.ds(SPLIT0, SPLIT1)], sem).start()


def _wait(b, buf, sem, tab_hbm, idx_v):
    off = b * L
    pltpu.make_async_copy(
        tab_hbm.at[idx_v.at[pl.ds(off, SPLIT0)]],
        buf.at[pl.ds(0, SPLIT0)], sem).wait()
    pltpu.make_async_copy(
        tab_hbm.at[idx_v.at[pl.ds(off + SPLIT0, SPLIT1)]],
        buf.at[pl.ds(SPLIT0, SPLIT1)], sem).wait()


def _accum_and_max(buf, macc):
    def lbody(l, accs):
        return tuple(a + buf[l, pl.ds(ci * 16, 16)]
                     for ci, a in enumerate(accs))
    zero = jnp.zeros((16,), jnp.float32)
    accs = lax.fori_loop(0, L, lbody, (zero,) * NCHUNK, unroll=4)
    for ci in range(NCHUNK):
        sl = pl.ds(ci * 16, 16)
        macc[sl] = jnp.maximum(macc[sl], accs[ci])


@functools.partial(
    pl.kernel,
    out_type=jax.ShapeDtypeStruct((NW, DP), jnp.float32),
    mesh=plsc.VectorSubcoreMesh(core_axis_name="c", subcore_axis_name="s"),
    scratch_types=[
        pltpu.VMEM((IPW,), jnp.int32),
        pltpu.VMEM((L, DP), jnp.float32),
        pltpu.VMEM((L, DP), jnp.float32),
        pltpu.VMEM((DP,), jnp.float32),
        pltpu.SemaphoreType.DMA,
        pltpu.SemaphoreType.DMA,
    ],
)
def _sc_gather_sum_max(xf_hbm, tab_hbm, part_hbm,
                       idx_v, buf0, buf1, macc, sem0, sem1):
    wid = lax.axis_index("s") * 2 + lax.axis_index("c")
    pltpu.sync_copy(xf_hbm.at[pl.ds(wid * IPW, IPW)], idx_v)

    neg = jnp.full((16,), -1e30, jnp.float32)
    for ci in range(NCHUNK):
        macc[pl.ds(ci * 16, 16)] = neg

    _issue(0, buf0, sem0, tab_hbm, idx_v)
    _issue(1, buf1, sem1, tab_hbm, idx_v)

    def pbody(p, carry):
        b0 = 2 * p
        _wait(b0, buf0, sem0, tab_hbm, idx_v)
        _accum_and_max(buf0, macc)

        @pl.when(p < BPW // 2 - 1)
        def _():
            _issue(b0 + 2, buf0, sem0, tab_hbm, idx_v)

        _wait(b0 + 1, buf1, sem1, tab_hbm, idx_v)
        _accum_and_max(buf1, macc)

        @pl.when(p < BPW // 2 - 1)
        def _():
            _issue(b0 + 3, buf1, sem1, tab_hbm, idx_v)

        return carry

    lax.fori_loop(0, BPW // 2, pbody, 0)
    pltpu.sync_copy(macc, part_hbm.at[wid])


VBLK = 16384
VOCAB = 1000000


def _tc_transpose_body(tin_ref, tout_ref):
    tout_ref[:, :D] = tin_ref[...].T


_tc_transpose = pl.pallas_call(
    _tc_transpose_body,
    grid=(pl.cdiv(VOCAB, VBLK),),
    in_specs=[pl.BlockSpec((D, VBLK), lambda g: (0, g))],
    out_specs=pl.BlockSpec((VBLK, DP), lambda g: (g, 0)),
    out_shape=jax.ShapeDtypeStruct((VOCAB, DP), jnp.float32),
)


def _tc_classifier_body(part_ref, w_ref, b_ref, y_ref, pred_ref, loss_ref):
    part = part_ref[...]
    pm = jnp.max(part, axis=0, keepdims=True)
    pm100 = pm[:, :D]
    pm100 = jnp.where(pm100 >= 0, pm100, 0.01 * pm100)
    logits = jnp.sum(w_ref[...] * pm100, axis=1, keepdims=True) + b_ref[...]
    m = jnp.max(logits, axis=0, keepdims=True)
    e = jnp.exp(logits - m)
    pred = e / jnp.sum(e, axis=0, keepdims=True)
    pred_ref[...] = pred
    pm2 = jnp.max(pred, axis=0, keepdims=True)
    e2 = jnp.exp(pred - pm2)
    lse = jnp.log(jnp.sum(e2, axis=0, keepdims=True)) + pm2
    log_probs = pred - lse
    loss_ref[...] = -jnp.sum(y_ref[...] * log_probs, axis=0, keepdims=True)


_tc_classifier = pl.pallas_call(
    _tc_classifier_body,
    out_shape=(jax.ShapeDtypeStruct((4, 1), jnp.float32),
               jax.ShapeDtypeStruct((1, 1), jnp.float32)),
)


def kernel(x, y, E_td, w_out, b_out):
    tab = _tc_transpose(E_td.T)
    part = _sc_gather_sum_max(x.reshape(-1), tab)
    pred2, loss2 = _tc_classifier(part, w_out, b_out[:, None], y[:, None])
    return pred2[:, 0], loss2[0, 0]


# speedup vs baseline: 5.4260x; 1.0038x over previous
import functools

import jax
import jax.numpy as jnp
from jax import lax
from jax.experimental import pallas as pl
from jax.experimental.pallas import tpu as pltpu
from jax.experimental.pallas import tpu_sc as plsc

D = 100
DP = 128
NCHUNK = 8
B = 4096
L = 200
NW = 32
BPW = B // NW
IPW = BPW * L
SPLIT0 = 96
SPLIT1 = L - SPLIT0


def _issue(b, buf, sem, tab_hbm, idx_v):
    off = b * L
    pltpu.make_async_copy(
        tab_hbm.at[idx_v.at[pl.ds(off, SPLIT0)]],
        buf.at[pl.ds(0, SPLIT0)], sem).start()
    pltpu.make_async_copy(
        tab_hbm.at[idx_v.at[pl.ds(off + SPLIT0, SPLIT1)]],
        buf.at[pl.ds(SPLIT0, SPLIT1)], sem).start()


def _wait(b, buf, sem, tab_hbm, idx_v):
    off = b * L
    pltpu.make_async_copy(
        tab_hbm.at[idx_v.at[pl.ds(off, SPLIT0)]],
        buf.at[pl.ds(0, SPLIT0)], sem).wait()
    pltpu.make_async_copy(
        tab_hbm.at[idx_v.at[pl.ds(off + SPLIT0, SPLIT1)]],
        buf.at[pl.ds(SPLIT0, SPLIT1)], sem).wait()


def _accum_and_max(buf, macc):
    def lbody(l, accs):
        return tuple(a + buf[l, pl.ds(ci * 16, 16)]
                     for ci, a in enumerate(accs))
    zero = jnp.zeros((16,), jnp.float32)
    accs = lax.fori_loop(0, L, lbody, (zero,) * NCHUNK, unroll=4)
    for ci in range(NCHUNK):
        sl = pl.ds(ci * 16, 16)
        macc[sl] = jnp.maximum(macc[sl], accs[ci])


@functools.partial(
    pl.kernel,
    out_type=jax.ShapeDtypeStruct((NW, DP), jnp.float32),
    mesh=plsc.VectorSubcoreMesh(core_axis_name="c", subcore_axis_name="s"),
    scratch_types=[
        pltpu.VMEM((IPW,), jnp.int32),
        pltpu.VMEM((L, DP), jnp.float32),
        pltpu.VMEM((L, DP), jnp.float32),
        pltpu.VMEM((DP,), jnp.float32),
        pltpu.SemaphoreType.DMA,
        pltpu.SemaphoreType.DMA,
    ],
)
def _sc_gather_sum_max(xf_hbm, tab_hbm, part_hbm,
                       idx_v, buf0, buf1, macc, sem0, sem1):
    wid = lax.axis_index("s") * 2 + lax.axis_index("c")
    pltpu.sync_copy(xf_hbm.at[pl.ds(wid * IPW, IPW)], idx_v)

    neg = jnp.full((16,), -1e30, jnp.float32)
    for ci in range(NCHUNK):
        macc[pl.ds(ci * 16, 16)] = neg

    _issue(0, buf0, sem0, tab_hbm, idx_v)
    _issue(1, buf1, sem1, tab_hbm, idx_v)

    def pbody(p, carry):
        b0 = 2 * p
        _wait(b0, buf0, sem0, tab_hbm, idx_v)
        _accum_and_max(buf0, macc)

        @pl.when(p < BPW // 2 - 1)
        def _():
            _issue(b0 + 2, buf0, sem0, tab_hbm, idx_v)

        _wait(b0 + 1, buf1, sem1, tab_hbm, idx_v)
        _accum_and_max(buf1, macc)

        @pl.when(p < BPW // 2 - 1)
        def _():
            _issue(b0 + 3, buf1, sem1, tab_hbm, idx_v)

        return carry

    lax.fori_loop(0, BPW // 2, pbody, 0)
    pltpu.sync_copy(macc, part_hbm.at[wid])


VBLK = 24576
VOCAB = 1000000


def _tc_transpose_body(tin_ref, tout_ref):
    tout_ref[:, :D] = tin_ref[...].T


_tc_transpose = pl.pallas_call(
    _tc_transpose_body,
    grid=(pl.cdiv(VOCAB, VBLK),),
    in_specs=[pl.BlockSpec((D, VBLK), lambda g: (0, g))],
    out_specs=pl.BlockSpec((VBLK, DP), lambda g: (g, 0)),
    out_shape=jax.ShapeDtypeStruct((VOCAB, DP), jnp.float32),
)


def _tc_classifier_body(part_ref, w_ref, b_ref, y_ref, pred_ref, loss_ref):
    part = part_ref[...]
    pm = jnp.max(part, axis=0, keepdims=True)
    pm100 = pm[:, :D]
    pm100 = jnp.where(pm100 >= 0, pm100, 0.01 * pm100)
    logits = jnp.sum(w_ref[...] * pm100, axis=1, keepdims=True) + b_ref[...]
    m = jnp.max(logits, axis=0, keepdims=True)
    e = jnp.exp(logits - m)
    pred = e / jnp.sum(e, axis=0, keepdims=True)
    pred_ref[...] = pred
    pm2 = jnp.max(pred, axis=0, keepdims=True)
    e2 = jnp.exp(pred - pm2)
    lse = jnp.log(jnp.sum(e2, axis=0, keepdims=True)) + pm2
    log_probs = pred - lse
    loss_ref[...] = -jnp.sum(y_ref[...] * log_probs, axis=0, keepdims=True)


_tc_classifier = pl.pallas_call(
    _tc_classifier_body,
    out_shape=(jax.ShapeDtypeStruct((4, 1), jnp.float32),
               jax.ShapeDtypeStruct((1, 1), jnp.float32)),
)


def kernel(x, y, E_td, w_out, b_out):
    tab = _tc_transpose(E_td.T)
    part = _sc_gather_sum_max(x.reshape(-1), tab)
    pred2, loss2 = _tc_classifier(part, w_out, b_out[:, None], y[:, None])
    return pred2[:, 0], loss2[0, 0]


# speedup vs baseline: 5.8769x; 1.0831x over previous
import functools

import jax
import jax.numpy as jnp
from jax import lax
from jax.experimental import pallas as pl
from jax.experimental.pallas import tpu as pltpu
from jax.experimental.pallas import tpu_sc as plsc

D = 100
DP = 128
NCHUNK = 8
B = 4096
L = 200
NW = 32
BPW = B // NW
IPW = BPW * L
SPLIT0 = 96
SPLIT1 = L - SPLIT0


def _issue(b, buf, sem, tab_hbm, idx_v):
    off = b * L
    pltpu.make_async_copy(
        tab_hbm.at[idx_v.at[pl.ds(off, SPLIT0)]],
        buf.at[pl.ds(0, SPLIT0)], sem).start()
    pltpu.make_async_copy(
        tab_hbm.at[idx_v.at[pl.ds(off + SPLIT0, SPLIT1)]],
        buf.at[pl.ds(SPLIT0, SPLIT1)], sem).start()


def _wait(b, buf, sem, tab_hbm, idx_v):
    off = b * L
    pltpu.make_async_copy(
        tab_hbm.at[idx_v.at[pl.ds(off, SPLIT0)]],
        buf.at[pl.ds(0, SPLIT0)], sem).wait()
    pltpu.make_async_copy(
        tab_hbm.at[idx_v.at[pl.ds(off + SPLIT0, SPLIT1)]],
        buf.at[pl.ds(SPLIT0, SPLIT1)], sem).wait()


def _accum_and_max(buf, macc):
    def lbody(l, accs):
        return tuple(a + buf[l, pl.ds(ci * 16, 16)]
                     for ci, a in enumerate(accs))
    zero = jnp.zeros((16,), jnp.float32)
    accs = lax.fori_loop(0, L, lbody, (zero,) * NCHUNK, unroll=4)
    for ci in range(NCHUNK):
        sl = pl.ds(ci * 16, 16)
        macc[sl] = jnp.maximum(macc[sl], accs[ci])


@functools.partial(
    pl.kernel,
    out_type=jax.ShapeDtypeStruct((NW, DP), jnp.float32),
    mesh=plsc.VectorSubcoreMesh(core_axis_name="c", subcore_axis_name="s"),
    scratch_types=[
        pltpu.VMEM((IPW,), jnp.int32),
        pltpu.VMEM((L, DP), jnp.float32),
        pltpu.VMEM((L, DP), jnp.float32),
        pltpu.VMEM((L, DP), jnp.float32),
        pltpu.VMEM((DP,), jnp.float32),
        pltpu.SemaphoreType.DMA,
        pltpu.SemaphoreType.DMA,
        pltpu.SemaphoreType.DMA,
    ],
)
def _sc_gather_sum_max(xf_hbm, tab_hbm, part_hbm,
                       idx_v, buf0, buf1, buf2, macc, sem0, sem1, sem2):
    wid = lax.axis_index("s") * 2 + lax.axis_index("c")
    pltpu.sync_copy(xf_hbm.at[pl.ds(wid * IPW, IPW)], idx_v)

    neg = jnp.full((16,), -1e30, jnp.float32)
    for ci in range(NCHUNK):
        macc[pl.ds(ci * 16, 16)] = neg

    bufs = (buf0, buf1, buf2)
    sems = (sem0, sem1, sem2)
    for q in range(3):
        _issue(q, bufs[q], sems[q], tab_hbm, idx_v)

    def pbody(p, carry):
        for q in range(3):
            b = 3 * p + q
            _wait(b, bufs[q], sems[q], tab_hbm, idx_v)
            _accum_and_max(bufs[q], macc)

            @pl.when(b + 3 < BPW)
            def _():
                _issue(b + 3, bufs[q], sems[q], tab_hbm, idx_v)
        return carry

    lax.fori_loop(0, BPW // 3, pbody, 0)
    for b in range(BPW - BPW % 3, BPW):
        q = b % 3
        _wait(b, bufs[q], sems[q], tab_hbm, idx_v)
        _accum_and_max(bufs[q], macc)

    pltpu.sync_copy(macc, part_hbm.at[wid])


VBLK = 24576
VOCAB = 1000000


def _tc_transpose_body(tin_ref, tout_ref):
    tout_ref[:, :D] = tin_ref[...].T


_tc_transpose = pl.pallas_call(
    _tc_transpose_body,
    grid=(pl.cdiv(VOCAB, VBLK),),
    in_specs=[pl.BlockSpec((D, VBLK), lambda g: (0, g))],
    out_specs=pl.BlockSpec((VBLK, DP), lambda g: (g, 0)),
    out_shape=jax.ShapeDtypeStruct((VOCAB, DP), jnp.float32),
)


def _tc_classifier_body(part_ref, w_ref, b_ref, y_ref, pred_ref, loss_ref):
    part = part_ref[...]
    pm = jnp.max(part, axis=0, keepdims=True)
    pm100 = pm[:, :D]
    pm100 = jnp.where(pm100 >= 0, pm100, 0.01 * pm100)
    logits = jnp.sum(w_ref[...] * pm100, axis=1, keepdims=True) + b_ref[...]
    m = jnp.max(logits, axis=0, keepdims=True)
    e = jnp.exp(logits - m)
    pred = e / jnp.sum(e, axis=0, keepdims=True)
    pred_ref[...] = pred
    pm2 = jnp.max(pred, axis=0, keepdims=True)
    e2 = jnp.exp(pred - pm2)
    lse = jnp.log(jnp.sum(e2, axis=0, keepdims=True)) + pm2
    log_probs = pred - lse
    loss_ref[...] = -jnp.sum(y_ref[...] * log_probs, axis=0, keepdims=True)


_tc_classifier = pl.pallas_call(
    _tc_classifier_body,
    out_shape=(jax.ShapeDtypeStruct((4, 1), jnp.float32),
               jax.ShapeDtypeStruct((1, 1), jnp.float32)),
)


def kernel(x, y, E_td, w_out, b_out):
    tab = _tc_transpose(E_td.T)
    part = _sc_gather_sum_max(x.reshape(-1), tab)
    pred2, loss2 = _tc_classifier(part, w_out, b_out[:, None], y[:, None])
    return pred2[:, 0], loss2[0, 0]
